# baseline (speedup 1.0000x reference)
import functools

import jax
import jax.numpy as jnp
from jax import lax
from jax.experimental import pallas as pl
from jax.experimental.pallas import tpu as pltpu
from jax.experimental.pallas import tpu_sc as plsc

NC = 2
NS = 16
NW = NC * NS
K = 256


def _pad_count(n, m):
    return (n + m - 1) // m * m


def _sc_degree(edge, zeros_np, NP, E):
    EPT = E // NW
    NJF = EPT // K
    REM = EPT - NJF * K

    @functools.partial(
        pl.kernel,
        out_type=jax.ShapeDtypeStruct((NC, NP), jnp.float32),
        mesh=plsc.VectorSubcoreMesh(core_axis_name="c", subcore_axis_name="s"),
        compiler_params=pltpu.CompilerParams(use_tc_tiling_on_sc=False),
        scratch_types=[
            pltpu.VMEM((EPT,), jnp.int32),
            pltpu.VMEM((K,), jnp.float32),
            pltpu.VMEM_SHARED((NP,), jnp.float32),
        ],
    )
    def deg_kernel(edge_hbm, zero_hbm, out_hbm, dst_v, ones_v, acc_sh):
        c = lax.axis_index("c")
        s = lax.axis_index("s")
        w = s * NC + c
        rpt = NP // NS
        r0 = s * rpt
        for i in range(K // 16):
            ones_v[pl.ds(i * 16, 16)] = jnp.ones((16,), jnp.float32)
        pltpu.sync_copy(zero_hbm.at[pl.ds(r0, rpt)], acc_sh.at[pl.ds(r0, rpt)])
        pltpu.sync_copy(edge_hbm.at[1, pl.ds(w * EPT, EPT)], dst_v)
        plsc.subcore_barrier()

        def step(j, carry):
            pltpu.sync_copy(ones_v, acc_sh.at[dst_v.at[pl.ds(j * K, K)]], add=True)
            return carry

        lax.fori_loop(0, NJF, step, 0)
        if REM:
            pltpu.sync_copy(ones_v.at[pl.ds(0, REM)],
                            acc_sh.at[dst_v.at[pl.ds(NJF * K, REM)]], add=True)
        plsc.subcore_barrier()
        pltpu.sync_copy(acc_sh.at[pl.ds(r0, rpt)], out_hbm.at[c, pl.ds(r0, rpt)])

    return deg_kernel(edge, zeros_np)


def _sc_msgpass(g, edge, zeros_npc, NP, E, C):
    NBUF = 4
    EPT = E // NW
    NJF = EPT // K
    REM = EPT - NJF * K
    G = NJF // NBUF
    TAIL = NJF - G * NBUF

    @functools.partial(
        pl.kernel,
        out_type=jax.ShapeDtypeStruct((NC, NP, C), jnp.float32),
        mesh=plsc.VectorSubcoreMesh(core_axis_name="c", subcore_axis_name="s"),
        compiler_params=pltpu.CompilerParams(use_tc_tiling_on_sc=False),
        scratch_types=[
            pltpu.VMEM((EPT,), jnp.int32),
            pltpu.VMEM((EPT,), jnp.int32),
            [pltpu.VMEM((K, C), jnp.float32) for _ in range(NBUF)],
            pltpu.VMEM_SHARED((NP, C), jnp.float32),
            [pltpu.SemaphoreType.DMA for _ in range(NBUF)],
            [pltpu.SemaphoreType.DMA for _ in range(NBUF)],
        ],
    )
    def mp_kernel(g_hbm, edge_hbm, zero_hbm, out_hbm,
                  src_v, dst_v, rows, acc_sh, sems, ssems):
        c = lax.axis_index("c")
        s = lax.axis_index("s")
        w = s * NC + c
        rpt = NP // NS
        r0 = s * rpt
        pltpu.sync_copy(zero_hbm.at[pl.ds(r0, rpt)], acc_sh.at[pl.ds(r0, rpt)])
        pltpu.sync_copy(edge_hbm.at[0, pl.ds(w * EPT, EPT)], src_v)
        pltpu.sync_copy(edge_hbm.at[1, pl.ds(w * EPT, EPT)], dst_v)
        plsc.subcore_barrier()

        def src_at(j):
            return src_v.at[pl.ds(j * K, K)]

        def dst_at(j):
            return dst_v.at[pl.ds(j * K, K)]

        for b in range(NBUF):
            pltpu.async_copy(g_hbm.at[src_at(b)], rows[b], sems[b])

        def step(t2, carry):
            for b in range(NBUF):
                j = NBUF * t2 + b
                pltpu.make_async_copy(g_hbm.at[src_at(j)], rows[b], sems[b]).wait()
                pltpu.async_copy(rows[b], acc_sh.at[dst_at(j)], ssems[b], add=True)
            for b in range(NBUF):
                j = NBUF * t2 + b
                pltpu.make_async_copy(rows[b], acc_sh.at[dst_at(j)], ssems[b]).wait()
                pltpu.async_copy(g_hbm.at[src_at(j + NBUF)], rows[b], sems[b])
            return carry

        lax.fori_loop(0, G - 1, step, 0)
        for b in range(NBUF):
            j = NBUF * (G - 1) + b
            pltpu.make_async_copy(g_hbm.at[src_at(j)], rows[b], sems[b]).wait()
            pltpu.async_copy(rows[b], acc_sh.at[dst_at(j)], ssems[b], add=True)
        for b in range(NBUF):
            j = NBUF * (G - 1) + b
            pltpu.make_async_copy(rows[b], acc_sh.at[dst_at(j)], ssems[b]).wait()
        for t in range(TAIL):
            j = G * NBUF + t
            pltpu.async_copy(g_hbm.at[src_at(j)], rows[0], sems[0]).wait()
            pltpu.sync_copy(rows[0], acc_sh.at[dst_at(j)], add=True)
        if REM:
            e0 = NJF * K
            pltpu.async_copy(g_hbm.at[src_v.at[pl.ds(e0, REM)]],
                             rows[0].at[pl.ds(0, REM)], sems[0]).wait()
            pltpu.sync_copy(rows[0].at[pl.ds(0, REM)],
                            acc_sh.at[dst_v.at[pl.ds(e0, REM)]], add=True)
        plsc.subcore_barrier()
        pltpu.sync_copy(acc_sh.at[pl.ds(r0, rpt)], out_hbm.at[c, pl.ds(r0, rpt)])

    return mp_kernel(g, edge, zeros_npc)


BN = 2000


def _tc1_body(x_ref, w1_ref, degt_ref, g1_ref, dinv_ref):
    deg = degt_ref[:, 0:1] + degt_ref[:, 1:2] + 1.0
    dinv = lax.rsqrt(jnp.clip(deg, 1.0, None))
    h = jnp.dot(x_ref[...], w1_ref[...], preferred_element_type=jnp.float32)
    g1_ref[...] = h * dinv
    dinv_ref[...] = dinv


def _tc2_body(s1_ref, g1_ref, dinv_ref, b1_ref, w2_ref, g2_ref):
    dinv = dinv_ref[...]
    s = s1_ref[0] + s1_ref[1] + g1_ref[...]
    h1 = jnp.maximum(s * dinv + b1_ref[...], 0.0)
    g2_ref[...] = jnp.dot(h1, w2_ref[...], preferred_element_type=jnp.float32) * dinv


def _tc3_body(s2_ref, g2_ref, dinv_ref, b2_ref, batch_ref, wfc_ref, bfc_ref,
              out_ref, sums_scr, cnt_scr, *, num_graphs, nblocks):
    i = pl.program_id(0)
    h2 = (s2_ref[0] + s2_ref[1] + g2_ref[...]) * dinv_ref[...] + b2_ref[...]
    bn = h2.shape[0]
    gid = lax.broadcasted_iota(jnp.int32, (bn, num_graphs), 1)
    maskt = (batch_ref[...] == gid).astype(jnp.float32)
    dn = (((0,), (0,)), ((), ()))
    psum = lax.dot_general(maskt, h2, dn, preferred_element_type=jnp.float32)
    pcnt = lax.dot_general(maskt, jnp.ones((bn, 1), jnp.float32), dn,
                           preferred_element_type=jnp.float32)
    psum = psum

    @pl.when(i == 0)
    def _():
        sums_scr[...] = jnp.zeros_like(sums_scr)
        cnt_scr[...] = jnp.zeros_like(cnt_scr)

    sums_scr[...] += psum
    cnt_scr[...] += pcnt

    @pl.when(i == nblocks - 1)
    def _():
        pooled = sums_scr[...] / jnp.clip(cnt_scr[...], 1.0, None)
        logits = jnp.dot(pooled, wfc_ref[...], preferred_element_type=jnp.float32)
        logits = logits + bfc_ref[...]
        m = jnp.max(logits, axis=1, keepdims=True)
        sh = logits - m
        lse = jnp.log(jnp.sum(jnp.exp(sh), axis=1, keepdims=True))
        out_ref[...] = sh - lse


def kernel(x, edge_index, batch, W1, b1, W2, b2, Wfc, bfc):
    N, _ = x.shape
    E = edge_index.shape[1]
    H1 = W1.shape[1]
    H2 = W2.shape[1]
    num_classes = Wfc.shape[1]
    num_graphs = 64

    NP = _pad_count(N, NS * 16)
    edge = edge_index.astype(jnp.int32)

    batch2 = batch.astype(jnp.int32).reshape(N, 1)
    z1 = jnp.zeros((NP,), jnp.float32)
    zc1 = jnp.zeros((NP, H1), jnp.float32)
    zc2 = jnp.zeros((NP, H2), jnp.float32)

    degp = _sc_degree(edge, z1, NP, E)
    degt = degp.T

    bn = BN if N % BN == 0 else N
    nb = N // bn
    F = x.shape[1]

    g1, dinv = pl.pallas_call(
        _tc1_body,
        grid=(nb,),
        in_specs=[
            pl.BlockSpec((bn, F), lambda i: (i, 0)),
            pl.BlockSpec((F, H1), lambda i: (0, 0)),
            pl.BlockSpec((bn, NC), lambda i: (i, 0)),
        ],
        out_specs=[
            pl.BlockSpec((bn, H1), lambda i: (i, 0)),
            pl.BlockSpec((bn, 1), lambda i: (i, 0)),
        ],
        out_shape=[
            jax.ShapeDtypeStruct((N, H1), jnp.float32),
            jax.ShapeDtypeStruct((N, 1), jnp.float32),
        ],
    )(x, W1, degt)

    s1 = _sc_msgpass(g1, edge, zc1, NP, E, H1)

    g2 = pl.pallas_call(
        _tc2_body,
        grid=(nb,),
        in_specs=[
            pl.BlockSpec((NC, bn, H1), lambda i: (0, i, 0)),
            pl.BlockSpec((bn, H1), lambda i: (i, 0)),
            pl.BlockSpec((bn, 1), lambda i: (i, 0)),
            pl.BlockSpec((1, H1), lambda i: (0, 0)),
            pl.BlockSpec((H1, H2), lambda i: (0, 0)),
        ],
        out_specs=pl.BlockSpec((bn, H2), lambda i: (i, 0)),
        out_shape=jax.ShapeDtypeStruct((N, H2), jnp.float32),
    )(s1, g1, dinv, b1.reshape(1, H1), W2)

    s2 = _sc_msgpass(g2, edge, zc2, NP, E, H2)

    out = pl.pallas_call(
        functools.partial(_tc3_body, num_graphs=num_graphs, nblocks=nb),
        grid=(nb,),
        in_specs=[
            pl.BlockSpec((NC, bn, H2), lambda i: (0, i, 0)),
            pl.BlockSpec((bn, H2), lambda i: (i, 0)),
            pl.BlockSpec((bn, 1), lambda i: (i, 0)),
            pl.BlockSpec((1, H2), lambda i: (0, 0)),
            pl.BlockSpec((bn, 1), lambda i: (i, 0)),
            pl.BlockSpec((H2, num_classes), lambda i: (0, 0)),
            pl.BlockSpec((1, num_classes), lambda i: (0, 0)),
        ],
        out_specs=pl.BlockSpec((num_graphs, num_classes), lambda i: (0, 0)),
        out_shape=jax.ShapeDtypeStruct((num_graphs, num_classes), jnp.float32),
        scratch_shapes=[
            pltpu.VMEM((num_graphs, H2), jnp.float32),
            pltpu.VMEM((num_graphs, 1), jnp.float32),
        ],
    )(s2, g2, dinv, b2.reshape(1, H2), batch2, Wfc, bfc.reshape(1, num_classes))

    return out


# speedup vs baseline: 1.0061x; 1.0061x over previous
import functools

import jax
import jax.numpy as jnp
from jax import lax
from jax.experimental import pallas as pl
from jax.experimental.pallas import tpu as pltpu
from jax.experimental.pallas import tpu_sc as plsc

NC = 2
NS = 16
NW = NC * NS
K = 256


def _pad_count(n, m):
    return (n + m - 1) // m * m


def _sc_degree(edge, zeros_np, NP, E):
    EPT = E // NW
    NJF = EPT // K
    REM = EPT - NJF * K

    @functools.partial(
        pl.kernel,
        out_type=jax.ShapeDtypeStruct((NC, NP), jnp.float32),
        mesh=plsc.VectorSubcoreMesh(core_axis_name="c", subcore_axis_name="s"),
        compiler_params=pltpu.CompilerParams(use_tc_tiling_on_sc=False),
        scratch_types=[
            pltpu.VMEM((EPT,), jnp.int32),
            pltpu.VMEM((K,), jnp.float32),
            pltpu.VMEM_SHARED((NP,), jnp.float32),
        ],
    )
    def deg_kernel(edge_hbm, zero_hbm, out_hbm, dst_v, ones_v, acc_sh):
        c = lax.axis_index("c")
        s = lax.axis_index("s")
        w = s * NC + c
        rpt = NP // NS
        r0 = s * rpt
        for i in range(K // 16):
            ones_v[pl.ds(i * 16, 16)] = jnp.ones((16,), jnp.float32)
        pltpu.sync_copy(zero_hbm.at[pl.ds(r0, rpt)], acc_sh.at[pl.ds(r0, rpt)])
        pltpu.sync_copy(edge_hbm.at[1, pl.ds(w * EPT, EPT)], dst_v)
        plsc.subcore_barrier()

        def step(j, carry):
            pltpu.sync_copy(ones_v, acc_sh.at[dst_v.at[pl.ds(j * K, K)]], add=True)
            return carry

        lax.fori_loop(0, NJF, step, 0)
        if REM:
            pltpu.sync_copy(ones_v.at[pl.ds(0, REM)],
                            acc_sh.at[dst_v.at[pl.ds(NJF * K, REM)]], add=True)
        plsc.subcore_barrier()
        pltpu.sync_copy(acc_sh.at[pl.ds(r0, rpt)], out_hbm.at[c, pl.ds(r0, rpt)])

    return deg_kernel(edge, zeros_np)


def _sc_msgpass(g, edge, zeros_npc, NP, E, C):
    NBUF = 4
    EPT = E // NW
    NJF = EPT // K
    REM = EPT - NJF * K
    G = NJF // NBUF
    TAIL = NJF - G * NBUF

    @functools.partial(
        pl.kernel,
        out_type=jax.ShapeDtypeStruct((NC, NP, C), jnp.float32),
        mesh=plsc.VectorSubcoreMesh(core_axis_name="c", subcore_axis_name="s"),
        compiler_params=pltpu.CompilerParams(use_tc_tiling_on_sc=False),
        scratch_types=[
            pltpu.VMEM((EPT,), jnp.int32),
            pltpu.VMEM((EPT,), jnp.int32),
            [pltpu.VMEM((K, C), jnp.float32) for _ in range(NBUF)],
            pltpu.VMEM_SHARED((NP, C), jnp.float32),
            [pltpu.SemaphoreType.DMA for _ in range(NBUF)],
        ],
    )
    def mp_kernel(g_hbm, edge_hbm, zero_hbm, out_hbm,
                  src_v, dst_v, rows, acc_sh, sems):
        c = lax.axis_index("c")
        s = lax.axis_index("s")
        w = s * NC + c
        rpt = NP // NS
        r0 = s * rpt
        pltpu.sync_copy(zero_hbm.at[pl.ds(r0, rpt)], acc_sh.at[pl.ds(r0, rpt)])
        pltpu.sync_copy(edge_hbm.at[0, pl.ds(w * EPT, EPT)], src_v)
        pltpu.sync_copy(edge_hbm.at[1, pl.ds(w * EPT, EPT)], dst_v)
        plsc.subcore_barrier()

        def src_at(j):
            return src_v.at[pl.ds(j * K, K)]

        def dst_at(j):
            return dst_v.at[pl.ds(j * K, K)]

        for b in range(NBUF):
            pltpu.async_copy(g_hbm.at[src_at(b)], rows[b], sems[b])

        def step(t2, carry):
            for b in range(NBUF):
                j = NBUF * t2 + b
                pltpu.make_async_copy(g_hbm.at[src_at(j)], rows[b], sems[b]).wait()
                pltpu.sync_copy(rows[b], acc_sh.at[dst_at(j)], add=True)
                pltpu.async_copy(g_hbm.at[src_at(j + NBUF)], rows[b], sems[b])
            return carry

        lax.fori_loop(0, G - 1, step, 0)
        for b in range(NBUF):
            j = NBUF * (G - 1) + b
            pltpu.make_async_copy(g_hbm.at[src_at(j)], rows[b], sems[b]).wait()
            pltpu.sync_copy(rows[b], acc_sh.at[dst_at(j)], add=True)
        for t in range(TAIL):
            j = G * NBUF + t
            pltpu.async_copy(g_hbm.at[src_at(j)], rows[0], sems[0]).wait()
            pltpu.sync_copy(rows[0], acc_sh.at[dst_at(j)], add=True)
        if REM:
            e0 = NJF * K
            pltpu.async_copy(g_hbm.at[src_v.at[pl.ds(e0, REM)]],
                             rows[0].at[pl.ds(0, REM)], sems[0]).wait()
            pltpu.sync_copy(rows[0].at[pl.ds(0, REM)],
                            acc_sh.at[dst_v.at[pl.ds(e0, REM)]], add=True)
        plsc.subcore_barrier()
        pltpu.sync_copy(acc_sh.at[pl.ds(r0, rpt)], out_hbm.at[c, pl.ds(r0, rpt)])

    return mp_kernel(g, edge, zeros_npc)


BN = 2000


def _tc1a_body(x_ref, w1_ref, h1pre_ref):
    h1pre_ref[...] = jnp.dot(x_ref[...], w1_ref[...],
                             preferred_element_type=jnp.float32)


def _tc1b_body(h1pre_ref, degt_ref, g1_ref, dinv_ref):
    deg = degt_ref[:, 0:1] + degt_ref[:, 1:2] + 1.0
    dinv = lax.rsqrt(jnp.clip(deg, 1.0, None))
    g1_ref[...] = h1pre_ref[...] * dinv
    dinv_ref[...] = dinv


def _tc2_body(s1_ref, g1_ref, dinv_ref, b1_ref, w2_ref, g2_ref):
    dinv = dinv_ref[...]
    s = s1_ref[0] + s1_ref[1] + g1_ref[...]
    h1 = jnp.maximum(s * dinv + b1_ref[...], 0.0)
    g2_ref[...] = jnp.dot(h1, w2_ref[...], preferred_element_type=jnp.float32) * dinv


def _maskt_of(batch_col, bn, num_graphs):
    gid = lax.broadcasted_iota(jnp.int32, (bn, num_graphs), 1)
    return (batch_col == gid).astype(jnp.float32)


_DN0 = (((0,), (0,)), ((), ()))


def _tc3a_body(g2_ref, dinv_ref, batch_ref, base_ref, cnt_ref,
               base_scr, cnt_scr, *, num_graphs, nblocks):
    i = pl.program_id(0)
    t = g2_ref[...] * dinv_ref[...]
    bn = t.shape[0]
    maskt = _maskt_of(batch_ref[...], bn, num_graphs)
    psum = lax.dot_general(maskt, t, _DN0, preferred_element_type=jnp.float32)
    pcnt = lax.dot_general(maskt, jnp.ones((bn, 1), jnp.float32), _DN0,
                           preferred_element_type=jnp.float32)

    @pl.when(i == 0)
    def _():
        base_scr[...] = jnp.zeros_like(base_scr)
        cnt_scr[...] = jnp.zeros_like(cnt_scr)

    base_scr[...] += psum
    cnt_scr[...] += pcnt

    @pl.when(i == nblocks - 1)
    def _():
        base_ref[...] = base_scr[...]
        cnt_ref[...] = cnt_scr[...]


def _tc3b_body(s2_ref, dinv_ref, batch_ref, base_ref, cnt_ref, b2_ref,
               wfc_ref, bfc_ref, out_ref, sums_scr, *, num_graphs, nblocks):
    i = pl.program_id(0)
    t = (s2_ref[0] + s2_ref[1]) * dinv_ref[...]
    bn = t.shape[0]
    maskt = _maskt_of(batch_ref[...], bn, num_graphs)
    psum = lax.dot_general(maskt, t, _DN0, preferred_element_type=jnp.float32)

    @pl.when(i == 0)
    def _():
        sums_scr[...] = jnp.zeros_like(sums_scr)

    sums_scr[...] += psum

    @pl.when(i == nblocks - 1)
    def _():
        cnt = cnt_ref[...]
        sums = sums_scr[...] + base_ref[...] + cnt * b2_ref[...]
        pooled = sums / jnp.clip(cnt, 1.0, None)
        logits = jnp.dot(pooled, wfc_ref[...], preferred_element_type=jnp.float32)
        logits = logits + bfc_ref[...]
        m = jnp.max(logits, axis=1, keepdims=True)
        sh = logits - m
        lse = jnp.log(jnp.sum(jnp.exp(sh), axis=1, keepdims=True))
        out_ref[...] = sh - lse


def kernel(x, edge_index, batch, W1, b1, W2, b2, Wfc, bfc):
    N, _ = x.shape
    E = edge_index.shape[1]
    H1 = W1.shape[1]
    H2 = W2.shape[1]
    num_classes = Wfc.shape[1]
    num_graphs = 64

    NP = _pad_count(N, NS * 16)
    edge = edge_index.astype(jnp.int32)

    batch2 = batch.astype(jnp.int32).reshape(N, 1)
    z1 = jnp.zeros((NP,), jnp.float32)
    zc1 = jnp.zeros((NP, H1), jnp.float32)
    zc2 = jnp.zeros((NP, H2), jnp.float32)

    degp = _sc_degree(edge, z1, NP, E)
    degt = degp.T

    bn = BN if N % BN == 0 else N
    nb = N // bn
    F = x.shape[1]

    h1pre = pl.pallas_call(
        _tc1a_body,
        grid=(nb,),
        in_specs=[
            pl.BlockSpec((bn, F), lambda i: (i, 0)),
            pl.BlockSpec((F, H1), lambda i: (0, 0)),
        ],
        out_specs=pl.BlockSpec((bn, H1), lambda i: (i, 0)),
        out_shape=jax.ShapeDtypeStruct((N, H1), jnp.float32),
    )(x, W1)

    g1, dinv = pl.pallas_call(
        _tc1b_body,
        grid=(nb,),
        in_specs=[
            pl.BlockSpec((bn, H1), lambda i: (i, 0)),
            pl.BlockSpec((bn, NC), lambda i: (i, 0)),
        ],
        out_specs=[
            pl.BlockSpec((bn, H1), lambda i: (i, 0)),
            pl.BlockSpec((bn, 1), lambda i: (i, 0)),
        ],
        out_shape=[
            jax.ShapeDtypeStruct((N, H1), jnp.float32),
            jax.ShapeDtypeStruct((N, 1), jnp.float32),
        ],
    )(h1pre, degt)

    s1 = _sc_msgpass(g1, edge, zc1, NP, E, H1)

    g2 = pl.pallas_call(
        _tc2_body,
        grid=(nb,),
        in_specs=[
            pl.BlockSpec((NC, bn, H1), lambda i: (0, i, 0)),
            pl.BlockSpec((bn, H1), lambda i: (i, 0)),
            pl.BlockSpec((bn, 1), lambda i: (i, 0)),
            pl.BlockSpec((1, H1), lambda i: (0, 0)),
            pl.BlockSpec((H1, H2), lambda i: (0, 0)),
        ],
        out_specs=pl.BlockSpec((bn, H2), lambda i: (i, 0)),
        out_shape=jax.ShapeDtypeStruct((N, H2), jnp.float32),
    )(s1, g1, dinv, b1.reshape(1, H1), W2)

    s2 = _sc_msgpass(g2, edge, zc2, NP, E, H2)

    base, cnt = pl.pallas_call(
        functools.partial(_tc3a_body, num_graphs=num_graphs, nblocks=nb),
        grid=(nb,),
        in_specs=[
            pl.BlockSpec((bn, H2), lambda i: (i, 0)),
            pl.BlockSpec((bn, 1), lambda i: (i, 0)),
            pl.BlockSpec((bn, 1), lambda i: (i, 0)),
        ],
        out_specs=[
            pl.BlockSpec((num_graphs, H2), lambda i: (0, 0)),
            pl.BlockSpec((num_graphs, 1), lambda i: (0, 0)),
        ],
        out_shape=[
            jax.ShapeDtypeStruct((num_graphs, H2), jnp.float32),
            jax.ShapeDtypeStruct((num_graphs, 1), jnp.float32),
        ],
        scratch_shapes=[
            pltpu.VMEM((num_graphs, H2), jnp.float32),
            pltpu.VMEM((num_graphs, 1), jnp.float32),
        ],
    )(g2, dinv, batch2)

    out = pl.pallas_call(
        functools.partial(_tc3b_body, num_graphs=num_graphs, nblocks=nb),
        grid=(nb,),
        in_specs=[
            pl.BlockSpec((NC, bn, H2), lambda i: (0, i, 0)),
            pl.BlockSpec((bn, 1), lambda i: (i, 0)),
            pl.BlockSpec((bn, 1), lambda i: (i, 0)),
            pl.BlockSpec((num_graphs, H2), lambda i: (0, 0)),
            pl.BlockSpec((num_graphs, 1), lambda i: (0, 0)),
            pl.BlockSpec((1, H2), lambda i: (0, 0)),
            pl.BlockSpec((H2, num_classes), lambda i: (0, 0)),
            pl.BlockSpec((1, num_classes), lambda i: (0, 0)),
        ],
        out_specs=pl.BlockSpec((num_graphs, num_classes), lambda i: (0, 0)),
        out_shape=jax.ShapeDtypeStruct((num_graphs, num_classes), jnp.float32),
        scratch_shapes=[
            pltpu.VMEM((num_graphs, H2), jnp.float32),
        ],
    )(s2, dinv, batch2, base, cnt, b2.reshape(1, H2), Wfc,
      bfc.reshape(1, num_classes))

    return out


# speedup vs baseline: 1.0344x; 1.0281x over previous
import functools

import jax
import jax.numpy as jnp
from jax import lax
from jax.experimental import pallas as pl
from jax.experimental.pallas import tpu as pltpu
from jax.experimental.pallas import tpu_sc as plsc

NC = 2
NS = 16
NW = NC * NS
K = 256


def _pad_count(n, m):
    return (n + m - 1) // m * m


def _sc_degree(edge, zeros_np, NP, E):
    EPT = E // NW
    NJF = EPT // K
    REM = EPT - NJF * K

    @functools.partial(
        pl.kernel,
        out_type=jax.ShapeDtypeStruct((NC, NP), jnp.float32),
        mesh=plsc.VectorSubcoreMesh(core_axis_name="c", subcore_axis_name="s"),
        compiler_params=pltpu.CompilerParams(use_tc_tiling_on_sc=False),
        scratch_types=[
            pltpu.VMEM((EPT,), jnp.int32),
            pltpu.VMEM((K,), jnp.float32),
            pltpu.VMEM_SHARED((NP,), jnp.float32),
        ],
    )
    def deg_kernel(edge_hbm, zero_hbm, out_hbm, dst_v, ones_v, acc_sh):
        c = lax.axis_index("c")
        s = lax.axis_index("s")
        w = s * NC + c
        rpt = NP // NS
        r0 = s * rpt
        for i in range(K // 16):
            ones_v[pl.ds(i * 16, 16)] = jnp.ones((16,), jnp.float32)
        pltpu.sync_copy(zero_hbm.at[pl.ds(r0, rpt)], acc_sh.at[pl.ds(r0, rpt)])
        pltpu.sync_copy(edge_hbm.at[1, pl.ds(w * EPT, EPT)], dst_v)
        plsc.subcore_barrier()

        def step(j, carry):
            pltpu.sync_copy(ones_v, acc_sh.at[dst_v.at[pl.ds(j * K, K)]], add=True)
            return carry

        lax.fori_loop(0, NJF, step, 0)
        if REM:
            pltpu.sync_copy(ones_v.at[pl.ds(0, REM)],
                            acc_sh.at[dst_v.at[pl.ds(NJF * K, REM)]], add=True)
        plsc.subcore_barrier()
        pltpu.sync_copy(acc_sh.at[pl.ds(r0, rpt)], out_hbm.at[c, pl.ds(r0, rpt)])

    return deg_kernel(edge, zeros_np)


def _sc_msgpass(g, edge, zeros_npc, NP, E, C):
    NBUF = 6
    EPT = E // NW
    NJF = EPT // K
    REM = EPT - NJF * K
    G = NJF // NBUF
    TAIL = NJF - G * NBUF

    @functools.partial(
        pl.kernel,
        out_type=jax.ShapeDtypeStruct((NC, NP, C), jnp.float32),
        mesh=plsc.VectorSubcoreMesh(core_axis_name="c", subcore_axis_name="s"),
        compiler_params=pltpu.CompilerParams(use_tc_tiling_on_sc=False),
        scratch_types=[
            pltpu.VMEM((EPT,), jnp.int32),
            pltpu.VMEM((EPT,), jnp.int32),
            [pltpu.VMEM((K, C), jnp.float32) for _ in range(NBUF)],
            pltpu.VMEM_SHARED((NP, C), jnp.float32),
            [pltpu.SemaphoreType.DMA for _ in range(NBUF)],
        ],
    )
    def mp_kernel(g_hbm, edge_hbm, zero_hbm, out_hbm,
                  src_v, dst_v, rows, acc_sh, sems):
        c = lax.axis_index("c")
        s = lax.axis_index("s")
        w = s * NC + c
        rpt = NP // NS
        r0 = s * rpt
        pltpu.sync_copy(zero_hbm.at[pl.ds(r0, rpt)], acc_sh.at[pl.ds(r0, rpt)])
        pltpu.sync_copy(edge_hbm.at[0, pl.ds(w * EPT, EPT)], src_v)
        pltpu.sync_copy(edge_hbm.at[1, pl.ds(w * EPT, EPT)], dst_v)
        plsc.subcore_barrier()

        def src_at(j):
            return src_v.at[pl.ds(j * K, K)]

        def dst_at(j):
            return dst_v.at[pl.ds(j * K, K)]

        for b in range(NBUF):
            pltpu.async_copy(g_hbm.at[src_at(b)], rows[b], sems[b])

        def step(t2, carry):
            for b in range(NBUF):
                j = NBUF * t2 + b
                pltpu.make_async_copy(g_hbm.at[src_at(j)], rows[b], sems[b]).wait()
                pltpu.sync_copy(rows[b], acc_sh.at[dst_at(j)], add=True)
                pltpu.async_copy(g_hbm.at[src_at(j + NBUF)], rows[b], sems[b])
            return carry

        lax.fori_loop(0, G - 1, step, 0)
        for b in range(NBUF):
            j = NBUF * (G - 1) + b
            pltpu.make_async_copy(g_hbm.at[src_at(j)], rows[b], sems[b]).wait()
            pltpu.sync_copy(rows[b], acc_sh.at[dst_at(j)], add=True)
        for t in range(TAIL):
            j = G * NBUF + t
            pltpu.async_copy(g_hbm.at[src_at(j)], rows[0], sems[0]).wait()
            pltpu.sync_copy(rows[0], acc_sh.at[dst_at(j)], add=True)
        if REM:
            e0 = NJF * K
            pltpu.async_copy(g_hbm.at[src_v.at[pl.ds(e0, REM)]],
                             rows[0].at[pl.ds(0, REM)], sems[0]).wait()
            pltpu.sync_copy(rows[0].at[pl.ds(0, REM)],
                            acc_sh.at[dst_v.at[pl.ds(e0, REM)]], add=True)
        plsc.subcore_barrier()
        pltpu.sync_copy(acc_sh.at[pl.ds(r0, rpt)], out_hbm.at[c, pl.ds(r0, rpt)])

    return mp_kernel(g, edge, zeros_npc)


BN = 2000


def _tc1_body(x_ref, w1_ref, degt_ref, g1_ref, dinv_ref):
    deg = degt_ref[:, 0:1] + degt_ref[:, 1:2] + 1.0
    dinv = lax.rsqrt(jnp.clip(deg, 1.0, None))
    h = jnp.dot(x_ref[...], w1_ref[...], preferred_element_type=jnp.float32)
    g1_ref[...] = h * dinv
    dinv_ref[...] = dinv


def _tc2_body(s1_ref, g1_ref, dinv_ref, b1_ref, w2_ref, g2_ref):
    dinv = dinv_ref[...]
    s = s1_ref[0] + s1_ref[1] + g1_ref[...]
    h1 = jnp.maximum(s * dinv + b1_ref[...], 0.0)
    g2_ref[...] = jnp.dot(h1, w2_ref[...], preferred_element_type=jnp.float32) * dinv


def _maskt_of(batch_col, bn, num_graphs):
    gid = lax.broadcasted_iota(jnp.int32, (bn, num_graphs), 1)
    return (batch_col == gid).astype(jnp.float32)


_DN0 = (((0,), (0,)), ((), ()))


def _tc3_body(s2_ref, g2_ref, dinv_ref, b2_ref, batch_ref, wfc_ref, bfc_ref,
              out_ref, sums_scr, cnt_scr, *, num_graphs, nblocks):
    i = pl.program_id(0)
    h2 = (s2_ref[0] + s2_ref[1] + g2_ref[...]) * dinv_ref[...] + b2_ref[...]
    bn = h2.shape[0]
    maskt = _maskt_of(batch_ref[...], bn, num_graphs)
    psum = lax.dot_general(maskt, h2, _DN0, preferred_element_type=jnp.float32)
    pcnt = lax.dot_general(maskt, jnp.ones((bn, 1), jnp.float32), _DN0,
                           preferred_element_type=jnp.float32)

    @pl.when(i == 0)
    def _():
        sums_scr[...] = jnp.zeros_like(sums_scr)
        cnt_scr[...] = jnp.zeros_like(cnt_scr)

    sums_scr[...] += psum
    cnt_scr[...] += pcnt

    @pl.when(i == nblocks - 1)
    def _():
        pooled = sums_scr[...] / jnp.clip(cnt_scr[...], 1.0, None)
        logits = jnp.dot(pooled, wfc_ref[...], preferred_element_type=jnp.float32)
        logits = logits + bfc_ref[...]
        m = jnp.max(logits, axis=1, keepdims=True)
        sh = logits - m
        lse = jnp.log(jnp.sum(jnp.exp(sh), axis=1, keepdims=True))
        out_ref[...] = sh - lse


def kernel(x, edge_index, batch, W1, b1, W2, b2, Wfc, bfc):
    N, _ = x.shape
    E = edge_index.shape[1]
    H1 = W1.shape[1]
    H2 = W2.shape[1]
    num_classes = Wfc.shape[1]
    num_graphs = 64

    NP = _pad_count(N, NS * 16)
    edge = edge_index.astype(jnp.int32)

    batch2 = batch.astype(jnp.int32).reshape(N, 1)
    z1 = jnp.zeros((NP,), jnp.float32)
    zc1 = jnp.zeros((NP, H1), jnp.float32)
    zc2 = jnp.zeros((NP, H2), jnp.float32)

    degp = _sc_degree(edge, z1, NP, E)
    degt = degp.T

    bn = BN if N % BN == 0 else N
    nb = N // bn
    F = x.shape[1]

    g1, dinv = pl.pallas_call(
        _tc1_body,
        grid=(nb,),
        in_specs=[
            pl.BlockSpec((bn, F), lambda i: (i, 0)),
            pl.BlockSpec((F, H1), lambda i: (0, 0)),
            pl.BlockSpec((bn, NC), lambda i: (i, 0)),
        ],
        out_specs=[
            pl.BlockSpec((bn, H1), lambda i: (i, 0)),
            pl.BlockSpec((bn, 1), lambda i: (i, 0)),
        ],
        out_shape=[
            jax.ShapeDtypeStruct((N, H1), jnp.float32),
            jax.ShapeDtypeStruct((N, 1), jnp.float32),
        ],
    )(x, W1, degt)

    s1 = _sc_msgpass(g1, edge, zc1, NP, E, H1)

    g2 = pl.pallas_call(
        _tc2_body,
        grid=(nb,),
        in_specs=[
            pl.BlockSpec((NC, bn, H1), lambda i: (0, i, 0)),
            pl.BlockSpec((bn, H1), lambda i: (i, 0)),
            pl.BlockSpec((bn, 1), lambda i: (i, 0)),
            pl.BlockSpec((1, H1), lambda i: (0, 0)),
            pl.BlockSpec((H1, H2), lambda i: (0, 0)),
        ],
        out_specs=pl.BlockSpec((bn, H2), lambda i: (i, 0)),
        out_shape=jax.ShapeDtypeStruct((N, H2), jnp.float32),
    )(s1, g1, dinv, b1.reshape(1, H1), W2)

    s2 = _sc_msgpass(g2, edge, zc2, NP, E, H2)

    out = pl.pallas_call(
        functools.partial(_tc3_body, num_graphs=num_graphs, nblocks=nb),
        grid=(nb,),
        in_specs=[
            pl.BlockSpec((NC, bn, H2), lambda i: (0, i, 0)),
            pl.BlockSpec((bn, H2), lambda i: (i, 0)),
            pl.BlockSpec((bn, 1), lambda i: (i, 0)),
            pl.BlockSpec((1, H2), lambda i: (0, 0)),
            pl.BlockSpec((bn, 1), lambda i: (i, 0)),
            pl.BlockSpec((H2, num_classes), lambda i: (0, 0)),
            pl.BlockSpec((1, num_classes), lambda i: (0, 0)),
        ],
        out_specs=pl.BlockSpec((num_graphs, num_classes), lambda i: (0, 0)),
        out_shape=jax.ShapeDtypeStruct((num_graphs, num_classes), jnp.float32),
        scratch_shapes=[
            pltpu.VMEM((num_graphs, H2), jnp.float32),
            pltpu.VMEM((num_graphs, 1), jnp.float32),
        ],
    )(s2, g2, dinv, b2.reshape(1, H2), batch2, Wfc, bfc.reshape(1, num_classes))

    return out


# speedup vs baseline: 1.0469x; 1.0121x over previous
import functools

import jax
import jax.numpy as jnp
from jax import lax
from jax.experimental import pallas as pl
from jax.experimental.pallas import tpu as pltpu
from jax.experimental.pallas import tpu_sc as plsc

NC = 2
NS = 16
NW = NC * NS
K = 256


def _pad_count(n, m):
    return (n + m - 1) // m * m


def _sc_degree(edge, zeros_np, NP, E):
    EPT = E // NW
    NJF = EPT // K
    REM = EPT - NJF * K

    NSEM = 6
    G = NJF // NSEM
    TAIL = NJF - G * NSEM

    @functools.partial(
        pl.kernel,
        out_type=jax.ShapeDtypeStruct((NC, NP), jnp.float32),
        mesh=plsc.VectorSubcoreMesh(core_axis_name="c", subcore_axis_name="s"),
        compiler_params=pltpu.CompilerParams(use_tc_tiling_on_sc=False),
        scratch_types=[
            pltpu.VMEM((EPT,), jnp.int32),
            pltpu.VMEM((K,), jnp.float32),
            pltpu.VMEM_SHARED((NP,), jnp.float32),
            [pltpu.SemaphoreType.DMA for _ in range(NSEM)],
        ],
    )
    def deg_kernel(edge_hbm, zero_hbm, out_hbm, dst_v, ones_v, acc_sh, sems):
        c = lax.axis_index("c")
        s = lax.axis_index("s")
        w = s * NC + c
        rpt = NP // NS
        r0 = s * rpt
        for i in range(K // 16):
            ones_v[pl.ds(i * 16, 16)] = jnp.ones((16,), jnp.float32)
        pltpu.sync_copy(zero_hbm.at[pl.ds(r0, rpt)], acc_sh.at[pl.ds(r0, rpt)])
        pltpu.sync_copy(edge_hbm.at[1, pl.ds(w * EPT, EPT)], dst_v)
        plsc.subcore_barrier()

        def dst_at(j):
            return acc_sh.at[dst_v.at[pl.ds(j * K, K)]]

        for b in range(NSEM):
            pltpu.async_copy(ones_v, dst_at(b), sems[b], add=True)

        def step(t2, carry):
            for b in range(NSEM):
                j = NSEM * t2 + b
                pltpu.make_async_copy(ones_v, dst_at(j), sems[b]).wait()
                pltpu.async_copy(ones_v, dst_at(j + NSEM), sems[b], add=True)
            return carry

        lax.fori_loop(0, G - 1, step, 0)
        for b in range(NSEM):
            j = NSEM * (G - 1) + b
            pltpu.make_async_copy(ones_v, dst_at(j), sems[b]).wait()
        for t in range(TAIL):
            j = G * NSEM + t
            pltpu.sync_copy(ones_v, dst_at(j), add=True)
        if REM:
            pltpu.sync_copy(ones_v.at[pl.ds(0, REM)],
                            acc_sh.at[dst_v.at[pl.ds(NJF * K, REM)]], add=True)
        plsc.subcore_barrier()
        pltpu.sync_copy(acc_sh.at[pl.ds(r0, rpt)], out_hbm.at[c, pl.ds(r0, rpt)])

    return deg_kernel(edge, zeros_np)


def _sc_msgpass(g, edge, zeros_npc, NP, E, C):
    NBUF = 6
    EPT = E // NW
    NJF = EPT // K
    REM = EPT - NJF * K
    G = NJF // NBUF
    TAIL = NJF - G * NBUF

    @functools.partial(
        pl.kernel,
        out_type=jax.ShapeDtypeStruct((NC, NP, C), jnp.float32),
        mesh=plsc.VectorSubcoreMesh(core_axis_name="c", subcore_axis_name="s"),
        compiler_params=pltpu.CompilerParams(use_tc_tiling_on_sc=False),
        scratch_types=[
            pltpu.VMEM((EPT,), jnp.int32),
            pltpu.VMEM((EPT,), jnp.int32),
            [pltpu.VMEM((K, C), jnp.float32) for _ in range(NBUF)],
            pltpu.VMEM_SHARED((NP, C), jnp.float32),
            [pltpu.SemaphoreType.DMA for _ in range(NBUF)],
        ],
    )
    def mp_kernel(g_hbm, edge_hbm, zero_hbm, out_hbm,
                  src_v, dst_v, rows, acc_sh, sems):
        c = lax.axis_index("c")
        s = lax.axis_index("s")
        w = s * NC + c
        rpt = NP // NS
        r0 = s * rpt
        pltpu.sync_copy(zero_hbm.at[pl.ds(r0, rpt)], acc_sh.at[pl.ds(r0, rpt)])
        pltpu.sync_copy(edge_hbm.at[0, pl.ds(w * EPT, EPT)], src_v)
        pltpu.sync_copy(edge_hbm.at[1, pl.ds(w * EPT, EPT)], dst_v)
        plsc.subcore_barrier()

        def src_at(j):
            return src_v.at[pl.ds(j * K, K)]

        def dst_at(j):
            return dst_v.at[pl.ds(j * K, K)]

        for b in range(NBUF):
            pltpu.async_copy(g_hbm.at[src_at(b)], rows[b], sems[b])

        def step(t2, carry):
            for b in range(NBUF):
                j = NBUF * t2 + b
                pltpu.make_async_copy(g_hbm.at[src_at(j)], rows[b], sems[b]).wait()
                pltpu.sync_copy(rows[b], acc_sh.at[dst_at(j)], add=True)
                pltpu.async_copy(g_hbm.at[src_at(j + NBUF)], rows[b], sems[b])
            return carry

        lax.fori_loop(0, G - 1, step, 0)
        for b in range(NBUF):
            j = NBUF * (G - 1) + b
            pltpu.make_async_copy(g_hbm.at[src_at(j)], rows[b], sems[b]).wait()
            pltpu.sync_copy(rows[b], acc_sh.at[dst_at(j)], add=True)
        for t in range(TAIL):
            j = G * NBUF + t
            pltpu.async_copy(g_hbm.at[src_at(j)], rows[0], sems[0]).wait()
            pltpu.sync_copy(rows[0], acc_sh.at[dst_at(j)], add=True)
        if REM:
            e0 = NJF * K
            pltpu.async_copy(g_hbm.at[src_v.at[pl.ds(e0, REM)]],
                             rows[0].at[pl.ds(0, REM)], sems[0]).wait()
            pltpu.sync_copy(rows[0].at[pl.ds(0, REM)],
                            acc_sh.at[dst_v.at[pl.ds(e0, REM)]], add=True)
        plsc.subcore_barrier()
        pltpu.sync_copy(acc_sh.at[pl.ds(r0, rpt)], out_hbm.at[c, pl.ds(r0, rpt)])

    return mp_kernel(g, edge, zeros_npc)


BN = 2000


def _tc1_body(x_ref, w1_ref, degt_ref, g1_ref, dinv_ref):
    deg = degt_ref[:, 0:1] + degt_ref[:, 1:2] + 1.0
    dinv = lax.rsqrt(jnp.clip(deg, 1.0, None))
    h = jnp.dot(x_ref[...], w1_ref[...], preferred_element_type=jnp.float32)
    g1_ref[...] = h * dinv
    dinv_ref[...] = dinv


def _tc2_body(s1_ref, g1_ref, dinv_ref, b1_ref, w2_ref, g2_ref):
    dinv = dinv_ref[...]
    s = s1_ref[0] + s1_ref[1] + g1_ref[...]
    h1 = jnp.maximum(s * dinv + b1_ref[...], 0.0)
    g2_ref[...] = jnp.dot(h1, w2_ref[...], preferred_element_type=jnp.float32) * dinv


def _maskt_of(batch_col, bn, num_graphs):
    gid = lax.broadcasted_iota(jnp.int32, (bn, num_graphs), 1)
    return (batch_col == gid).astype(jnp.float32)


_DN0 = (((0,), (0,)), ((), ()))


def _tc3_body(s2_ref, g2_ref, dinv_ref, b2_ref, batch_ref, wfc_ref, bfc_ref,
              out_ref, sums_scr, cnt_scr, *, num_graphs, nblocks):
    i = pl.program_id(0)
    h2 = (s2_ref[0] + s2_ref[1] + g2_ref[...]) * dinv_ref[...] + b2_ref[...]
    bn = h2.shape[0]
    maskt = _maskt_of(batch_ref[...], bn, num_graphs)
    psum = lax.dot_general(maskt, h2, _DN0, preferred_element_type=jnp.float32)
    pcnt = lax.dot_general(maskt, jnp.ones((bn, 1), jnp.float32), _DN0,
                           preferred_element_type=jnp.float32)

    @pl.when(i == 0)
    def _():
        sums_scr[...] = jnp.zeros_like(sums_scr)
        cnt_scr[...] = jnp.zeros_like(cnt_scr)

    sums_scr[...] += psum
    cnt_scr[...] += pcnt

    @pl.when(i == nblocks - 1)
    def _():
        pooled = sums_scr[...] / jnp.clip(cnt_scr[...], 1.0, None)
        logits = jnp.dot(pooled, wfc_ref[...], preferred_element_type=jnp.float32)
        logits = logits + bfc_ref[...]
        m = jnp.max(logits, axis=1, keepdims=True)
        sh = logits - m
        lse = jnp.log(jnp.sum(jnp.exp(sh), axis=1, keepdims=True))
        out_ref[...] = sh - lse


def kernel(x, edge_index, batch, W1, b1, W2, b2, Wfc, bfc):
    N, _ = x.shape
    E = edge_index.shape[1]
    H1 = W1.shape[1]
    H2 = W2.shape[1]
    num_classes = Wfc.shape[1]
    num_graphs = 64

    NP = _pad_count(N, NS * 16)
    edge = edge_index.astype(jnp.int32)

    batch2 = batch.astype(jnp.int32).reshape(N, 1)
    z1 = jnp.zeros((NP,), jnp.float32)
    zc1 = jnp.zeros((NP, H1), jnp.float32)
    zc2 = jnp.zeros((NP, H2), jnp.float32)

    degp = _sc_degree(edge, z1, NP, E)
    degt = degp.T

    bn = BN if N % BN == 0 else N
    nb = N // bn
    F = x.shape[1]

    g1, dinv = pl.pallas_call(
        _tc1_body,
        grid=(nb,),
        in_specs=[
            pl.BlockSpec((bn, F), lambda i: (i, 0)),
            pl.BlockSpec((F, H1), lambda i: (0, 0)),
            pl.BlockSpec((bn, NC), lambda i: (i, 0)),
        ],
        out_specs=[
            pl.BlockSpec((bn, H1), lambda i: (i, 0)),
            pl.BlockSpec((bn, 1), lambda i: (i, 0)),
        ],
        out_shape=[
            jax.ShapeDtypeStruct((N, H1), jnp.float32),
            jax.ShapeDtypeStruct((N, 1), jnp.float32),
        ],
    )(x, W1, degt)

    s1 = _sc_msgpass(g1, edge, zc1, NP, E, H1)

    g2 = pl.pallas_call(
        _tc2_body,
        grid=(nb,),
        in_specs=[
            pl.BlockSpec((NC, bn, H1), lambda i: (0, i, 0)),
            pl.BlockSpec((bn, H1), lambda i: (i, 0)),
            pl.BlockSpec((bn, 1), lambda i: (i, 0)),
            pl.BlockSpec((1, H1), lambda i: (0, 0)),
            pl.BlockSpec((H1, H2), lambda i: (0, 0)),
        ],
        out_specs=pl.BlockSpec((bn, H2), lambda i: (i, 0)),
        out_shape=jax.ShapeDtypeStruct((N, H2), jnp.float32),
    )(s1, g1, dinv, b1.reshape(1, H1), W2)

    s2 = _sc_msgpass(g2, edge, zc2, NP, E, H2)

    out = pl.pallas_call(
        functools.partial(_tc3_body, num_graphs=num_graphs, nblocks=nb),
        grid=(nb,),
        in_specs=[
            pl.BlockSpec((NC, bn, H2), lambda i: (0, i, 0)),
            pl.BlockSpec((bn, H2), lambda i: (i, 0)),
            pl.BlockSpec((bn, 1), lambda i: (i, 0)),
            pl.BlockSpec((1, H2), lambda i: (0, 0)),
            pl.BlockSpec((bn, 1), lambda i: (i, 0)),
            pl.BlockSpec((H2, num_classes), lambda i: (0, 0)),
            pl.BlockSpec((1, num_classes), lambda i: (0, 0)),
        ],
        out_specs=pl.BlockSpec((num_graphs, num_classes), lambda i: (0, 0)),
        out_shape=jax.ShapeDtypeStruct((num_graphs, num_classes), jnp.float32),
        scratch_shapes=[
            pltpu.VMEM((num_graphs, H2), jnp.float32),
            pltpu.VMEM((num_graphs, 1), jnp.float32),
        ],
    )(s2, g2, dinv, b2.reshape(1, H2), batch2, Wfc, bfc.reshape(1, num_classes))

    return out
